# baseline (speedup 1.0000x reference)
import jax
import jax.numpy as jnp
from jax import lax
from jax.experimental import pallas as pl
from jax.experimental.pallas import tpu as pltpu
from jax.experimental.pallas import tpu_sc as plsc

N = 10000
D = 128
E = 320000
EPS = 1e-5

NC = 2
NS = 16
NW = NC * NS
EPT = E // NW
CHUNK = 80
NCH = EPT // CHUNK
NPAD = 10240
RPT = NPAD // NS

_MESH = plsc.VectorSubcoreMesh(core_axis_name="c", subcore_axis_name="s")
_SC_PARAMS = pltpu.CompilerParams(needs_layout_passes=False)


def _deg_body(dst_hbm, degp_hbm, idx_v, hist_v):
    c = lax.axis_index("c")
    s = lax.axis_index("s")
    wid = c * NS + s
    zero16 = jnp.zeros((16,), jnp.float32)

    @pl.loop(0, NPAD // 16, unroll=8)
    def _(i):
        hist_v[pl.ds(i * 16, 16)] = zero16

    pltpu.sync_copy(dst_hbm.at[pl.ds(wid * EPT, EPT)], idx_v)
    ones16 = jnp.ones((16,), jnp.float32)

    @pl.loop(0, EPT // 16, unroll=16)
    def _(i):
        idx = idx_v[pl.ds(i * 16, 16)]
        plsc.addupdate_scatter(hist_v, [idx], ones16)

    pltpu.sync_copy(hist_v, degp_hbm.at[wid])


_deg_kernel = pl.kernel(
    _deg_body,
    out_type=jax.ShapeDtypeStruct((NW, NPAD), jnp.float32),
    mesh=_MESH,
    compiler_params=_SC_PARAMS,
    scratch_types=[
        pltpu.VMEM((EPT,), jnp.int32),
        pltpu.VMEM((NPAD,), jnp.float32),
    ],
)


NBUF = 3
NFULL = NCH // NBUF


def _agg_body(g_hbm, src_hbm, dst_hbm, out_hbm, sidx_v,
              didx_bufs, rows_bufs, acc_sh, sems):
    c = lax.axis_index("c")
    s = lax.axis_index("s")
    wid = c * NS + s
    zero16 = jnp.zeros((16,), jnp.float32)
    rows_a, sem_a = rows_bufs[0], sems[0]

    sidx_copy = pltpu.async_copy(src_hbm.at[pl.ds(wid * EPT, EPT)], sidx_v, sem_a)

    @pl.loop(0, CHUNK, unroll=4)
    def _(r):
        @pl.loop(0, D // 16)
        def _(j):
            rows_a[r, pl.ds(j * 16, 16)] = zero16

    @pl.loop(0, RPT // CHUNK)
    def _(j):
        pltpu.sync_copy(rows_a, acc_sh.at[pl.ds(s * RPT + j * CHUNK, CHUNK)])

    sidx_copy.wait()

    def start_chunk(k, j):
        pltpu.async_copy(g_hbm.at[sidx_v.at[pl.ds(k * CHUNK, CHUNK)]],
                         rows_bufs[j], sems[j])
        pltpu.async_copy(dst_hbm.at[pl.ds(wid * EPT + k * CHUNK, CHUNK)],
                         didx_bufs[j], sems[j])

    def wait_chunk(j):
        pltpu.make_async_copy(g_hbm.at[pl.ds(0, CHUNK)], rows_bufs[j], sems[j]).wait()
        pltpu.make_async_copy(dst_hbm.at[pl.ds(0, CHUNK)], didx_bufs[j], sems[j]).wait()

    def scatter(j):
        pltpu.sync_copy(rows_bufs[j], acc_sh.at[didx_bufs[j]], add=True)

    for j in range(NBUF):
        start_chunk(j, j)

    plsc.subcore_barrier()

    @pl.loop(0, NFULL)
    def _(i):
        k = i * NBUF
        for j in range(NBUF):
            cidx = k + j
            wait_chunk(j)
            scatter(j)

            @pl.when(cidx + NBUF < NCH)
            def _():
                start_chunk(cidx + NBUF, j)

    for j in range(NCH % NBUF):
        wait_chunk(j)
        scatter(j)

    plsc.subcore_barrier()

    pltpu.sync_copy(acc_sh.at[pl.ds(s * RPT, RPT)], out_hbm.at[c, pl.ds(s * RPT, RPT)])


_agg_kernel = pl.kernel(
    _agg_body,
    out_type=jax.ShapeDtypeStruct((NC, NPAD, D), jnp.float32),
    mesh=_MESH,
    compiler_params=_SC_PARAMS,
    scratch_types=[
        pltpu.VMEM((EPT,), jnp.int32),
        tuple(pltpu.VMEM((CHUNK,), jnp.int32) for _ in range(NBUF)),
        tuple(pltpu.VMEM((CHUNK, D), jnp.float32) for _ in range(NBUF)),
        pltpu.VMEM_SHARED((NPAD, D), jnp.float32),
        tuple(pltpu.SemaphoreType.DMA for _ in range(NBUF)),
    ],
)

R = 1280
NBLK = NPAD // R


def _dinv_of(degp_blk):
    deg = jnp.sum(degp_blk, axis=0) + 1.0
    return lax.rsqrt(deg)[:, None]


def _mm_scale_body(x_ref, w_ref, degp_ref, g_ref):
    h = jnp.dot(x_ref[...], w_ref[...], preferred_element_type=jnp.float32)
    g_ref[...] = h * _dinv_of(degp_ref[...])


def _mm_scale(x, w, degp):
    return pl.pallas_call(
        _mm_scale_body,
        grid=(NBLK,),
        in_specs=[
            pl.BlockSpec((R, D), lambda i: (i, 0)),
            pl.BlockSpec((D, D), lambda i: (0, 0)),
            pl.BlockSpec((NW, R), lambda i: (0, i)),
        ],
        out_specs=pl.BlockSpec((R, D), lambda i: (i, 0)),
        out_shape=jax.ShapeDtypeStruct((N, D), jnp.float32),
    )(x, w, degp)


def _bn_mm_body(sp_ref, g_ref, degp_ref, b_ref, gamma_ref, beta_ref, w_ref,
                g2_ref, t_v, st_v):
    p = pl.program_id(0)
    i = pl.program_id(1)
    dinv = _dinv_of(degp_ref[...])

    @pl.when(p == 0)
    def _():
        t = (sp_ref[0] + sp_ref[1] + g_ref[...]) * dinv + b_ref[...]
        t_v[pl.ds(i * R, R), :] = t
        rows = i * R + lax.broadcasted_iota(jnp.int32, (R, 1), 0)
        tm = jnp.where(rows < N, t, 0.0)
        st = jnp.concatenate(
            [jnp.sum(tm, axis=0, keepdims=True),
             jnp.sum(tm * tm, axis=0, keepdims=True)],
            axis=0,
        )

        @pl.when(i == 0)
        def _():
            st_v[...] = st

        @pl.when(i > 0)
        def _():
            st_v[...] = st_v[...] + st

    @pl.when(p == 1)
    def _():
        mean = st_v[0:1, :] * (1.0 / N)
        var = st_v[1:2, :] * (1.0 / N) - mean * mean
        y = gamma_ref[...] * (t_v[pl.ds(i * R, R), :] - mean) * lax.rsqrt(var + EPS) + beta_ref[...]
        y = jnp.maximum(y, 0.0)
        h2 = jnp.dot(y, w_ref[...], preferred_element_type=jnp.float32)
        g2_ref[...] = h2 * dinv


def _bn_mm(sp, g, degp, b1, gamma, beta, w2):
    return pl.pallas_call(
        _bn_mm_body,
        grid=(2, NBLK),
        in_specs=[
            pl.BlockSpec((NC, R, D), lambda p, i: (0, i * (1 - p), 0)),
            pl.BlockSpec((R, D), lambda p, i: (i * (1 - p), 0)),
            pl.BlockSpec((NW, R), lambda p, i: (0, i)),
            pl.BlockSpec((1, D), lambda p, i: (0, 0)),
            pl.BlockSpec((1, D), lambda p, i: (0, 0)),
            pl.BlockSpec((1, D), lambda p, i: (0, 0)),
            pl.BlockSpec((D, D), lambda p, i: (0, 0)),
        ],
        out_specs=pl.BlockSpec((R, D), lambda p, i: (i, 0)),
        out_shape=jax.ShapeDtypeStruct((N, D), jnp.float32),
        scratch_shapes=[
            pltpu.VMEM((NPAD, D), jnp.float32),
            pltpu.VMEM((2, D), jnp.float32),
        ],
    )(sp, g, degp, b1, gamma, beta, w2)


def _final_body(sp_ref, g_ref, degp_ref, b_ref, out_ref):
    out_ref[...] = (sp_ref[0] + sp_ref[1] + g_ref[...]) * _dinv_of(degp_ref[...]) + b_ref[...]


def _final(sp, g, degp, b2):
    return pl.pallas_call(
        _final_body,
        grid=(NBLK,),
        in_specs=[
            pl.BlockSpec((NC, R, D), lambda i: (0, i, 0)),
            pl.BlockSpec((R, D), lambda i: (i, 0)),
            pl.BlockSpec((NW, R), lambda i: (0, i)),
            pl.BlockSpec((1, D), lambda i: (0, 0)),
        ],
        out_specs=pl.BlockSpec((R, D), lambda i: (i, 0)),
        out_shape=jax.ShapeDtypeStruct((N, D), jnp.float32),
    )(sp, g, degp, b2)


def kernel(x, edge_index, W1, b1, gamma, beta, W2, b2):
    src = edge_index[0].astype(jnp.int32)
    dst = edge_index[1].astype(jnp.int32)

    degp = _deg_kernel(dst)
    g1 = _mm_scale(x, W1, degp)
    sp1 = _agg_kernel(g1, src, dst)
    g2 = _bn_mm(sp1, g1, degp, b1.reshape(1, D), gamma.reshape(1, D),
                beta.reshape(1, D), W2)
    sp2 = _agg_kernel(g2, src, dst)
    return _final(sp2, g2, degp, b2.reshape(1, D))


# speedup vs baseline: 1.0303x; 1.0303x over previous
import jax
import jax.numpy as jnp
from jax import lax
from jax.experimental import pallas as pl
from jax.experimental.pallas import tpu as pltpu
from jax.experimental.pallas import tpu_sc as plsc

N = 10000
D = 128
E = 320000
EPS = 1e-5

NC = 2
NS = 16
NW = NC * NS
EPT = E // NW
CHUNK = 80
NCH = EPT // CHUNK
NPAD = 10240
RPT = NPAD // NS

_MESH = plsc.VectorSubcoreMesh(core_axis_name="c", subcore_axis_name="s")
_SC_PARAMS = pltpu.CompilerParams(needs_layout_passes=False)


def _deg_body(dst_hbm, degp_hbm, idx_v, hist_v):
    c = lax.axis_index("c")
    s = lax.axis_index("s")
    wid = c * NS + s
    zero16 = jnp.zeros((16,), jnp.float32)

    @pl.loop(0, NPAD // 16, unroll=8)
    def _(i):
        hist_v[pl.ds(i * 16, 16)] = zero16

    pltpu.sync_copy(dst_hbm.at[pl.ds(wid * EPT, EPT)], idx_v)
    ones16 = jnp.ones((16,), jnp.float32)

    @pl.loop(0, EPT // 16, unroll=16)
    def _(i):
        idx = idx_v[pl.ds(i * 16, 16)]
        plsc.addupdate_scatter(hist_v, [idx], ones16)

    pltpu.sync_copy(hist_v, degp_hbm.at[wid])


_deg_kernel = pl.kernel(
    _deg_body,
    out_type=jax.ShapeDtypeStruct((NW, NPAD), jnp.float32),
    mesh=_MESH,
    compiler_params=_SC_PARAMS,
    scratch_types=[
        pltpu.VMEM((EPT,), jnp.int32),
        pltpu.VMEM((NPAD,), jnp.float32),
    ],
)


NBUF = 3
NFULL = NCH // NBUF


def _agg_body(g_hbm, src_hbm, dst_hbm, out_hbm, sidx_v,
              didx_bufs, rows_bufs, acc_sh, sems):
    c = lax.axis_index("c")
    s = lax.axis_index("s")
    wid = c * NS + s
    zero16 = jnp.zeros((16,), jnp.float32)
    rows_a, sem_a = rows_bufs[0], sems[0]

    sidx_copy = pltpu.async_copy(src_hbm.at[pl.ds(wid * EPT, EPT)], sidx_v, sem_a)

    @pl.loop(0, CHUNK, unroll=4)
    def _(r):
        @pl.loop(0, D // 16)
        def _(j):
            rows_a[r, pl.ds(j * 16, 16)] = zero16

    @pl.loop(0, RPT // CHUNK)
    def _(j):
        pltpu.sync_copy(rows_a, acc_sh.at[pl.ds(s * RPT + j * CHUNK, CHUNK)])

    sidx_copy.wait()

    def start_chunk(k, j):
        pltpu.async_copy(g_hbm.at[sidx_v.at[pl.ds(k * CHUNK, CHUNK)]],
                         rows_bufs[j], sems[j])
        pltpu.async_copy(dst_hbm.at[pl.ds(wid * EPT + k * CHUNK, CHUNK)],
                         didx_bufs[j], sems[j])

    def wait_chunk(j):
        pltpu.make_async_copy(g_hbm.at[pl.ds(0, CHUNK)], rows_bufs[j], sems[j]).wait()
        pltpu.make_async_copy(dst_hbm.at[pl.ds(0, CHUNK)], didx_bufs[j], sems[j]).wait()

    def scatter(j):
        pltpu.sync_copy(rows_bufs[j], acc_sh.at[didx_bufs[j]], add=True)

    for j in range(NBUF):
        start_chunk(j, j)

    plsc.subcore_barrier()

    @pl.loop(0, NFULL)
    def _(i):
        k = i * NBUF
        for j in range(NBUF):
            cidx = k + j
            wait_chunk(j)
            scatter(j)

            @pl.when(cidx + NBUF < NCH)
            def _():
                start_chunk(cidx + NBUF, j)

    for j in range(NCH % NBUF):
        wait_chunk(j)
        scatter(j)

    plsc.subcore_barrier()

    pltpu.sync_copy(acc_sh.at[pl.ds(s * RPT, RPT)], out_hbm.at[c, pl.ds(s * RPT, RPT)])


_agg_kernel = pl.kernel(
    _agg_body,
    out_type=jax.ShapeDtypeStruct((NC, NPAD, D), jnp.float32),
    mesh=_MESH,
    compiler_params=_SC_PARAMS,
    scratch_types=[
        pltpu.VMEM((EPT,), jnp.int32),
        tuple(pltpu.VMEM((CHUNK,), jnp.int32) for _ in range(NBUF)),
        tuple(pltpu.VMEM((CHUNK, D), jnp.float32) for _ in range(NBUF)),
        pltpu.VMEM_SHARED((NPAD, D), jnp.float32),
        tuple(pltpu.SemaphoreType.DMA for _ in range(NBUF)),
    ],
)

R = 2560
NBLK = NPAD // R


def _dinv_of(degp_blk):
    deg = jnp.sum(degp_blk, axis=0) + 1.0
    return lax.rsqrt(deg)[:, None]


def _mm_scale_body(x_ref, w_ref, degp_ref, g_ref):
    h = jnp.dot(x_ref[...], w_ref[...], preferred_element_type=jnp.float32)
    g_ref[...] = h * _dinv_of(degp_ref[...])


def _mm_scale(x, w, degp):
    return pl.pallas_call(
        _mm_scale_body,
        grid=(NBLK,),
        in_specs=[
            pl.BlockSpec((R, D), lambda i: (i, 0)),
            pl.BlockSpec((D, D), lambda i: (0, 0)),
            pl.BlockSpec((NW, R), lambda i: (0, i)),
        ],
        out_specs=pl.BlockSpec((R, D), lambda i: (i, 0)),
        out_shape=jax.ShapeDtypeStruct((N, D), jnp.float32),
    )(x, w, degp)


def _bn_mm_body(sp_ref, g_ref, degp_ref, b_ref, gamma_ref, beta_ref, w_ref,
                g2_ref, t_v, st_v):
    p = pl.program_id(0)
    i = pl.program_id(1)
    dinv = _dinv_of(degp_ref[...])

    @pl.when(p == 0)
    def _():
        t = (sp_ref[0] + sp_ref[1] + g_ref[...]) * dinv + b_ref[...]
        t_v[pl.ds(i * R, R), :] = t
        rows = i * R + lax.broadcasted_iota(jnp.int32, (R, 1), 0)
        tm = jnp.where(rows < N, t, 0.0)
        st = jnp.concatenate(
            [jnp.sum(tm, axis=0, keepdims=True),
             jnp.sum(tm * tm, axis=0, keepdims=True)],
            axis=0,
        )

        @pl.when(i == 0)
        def _():
            st_v[...] = st

        @pl.when(i > 0)
        def _():
            st_v[...] = st_v[...] + st

    @pl.when(p == 1)
    def _():
        mean = st_v[0:1, :] * (1.0 / N)
        var = st_v[1:2, :] * (1.0 / N) - mean * mean
        y = gamma_ref[...] * (t_v[pl.ds(i * R, R), :] - mean) * lax.rsqrt(var + EPS) + beta_ref[...]
        y = jnp.maximum(y, 0.0)
        h2 = jnp.dot(y, w_ref[...], preferred_element_type=jnp.float32)
        g2_ref[...] = h2 * dinv


def _bn_mm(sp, g, degp, b1, gamma, beta, w2):
    return pl.pallas_call(
        _bn_mm_body,
        grid=(2, NBLK),
        in_specs=[
            pl.BlockSpec((NC, R, D), lambda p, i: (0, i * (1 - p), 0)),
            pl.BlockSpec((R, D), lambda p, i: (i * (1 - p), 0)),
            pl.BlockSpec((NW, R), lambda p, i: (0, i)),
            pl.BlockSpec((1, D), lambda p, i: (0, 0)),
            pl.BlockSpec((1, D), lambda p, i: (0, 0)),
            pl.BlockSpec((1, D), lambda p, i: (0, 0)),
            pl.BlockSpec((D, D), lambda p, i: (0, 0)),
        ],
        out_specs=pl.BlockSpec((R, D), lambda p, i: (i, 0)),
        out_shape=jax.ShapeDtypeStruct((N, D), jnp.float32),
        scratch_shapes=[
            pltpu.VMEM((NPAD, D), jnp.float32),
            pltpu.VMEM((2, D), jnp.float32),
        ],
    )(sp, g, degp, b1, gamma, beta, w2)


def _final_body(sp_ref, g_ref, degp_ref, b_ref, out_ref):
    out_ref[...] = (sp_ref[0] + sp_ref[1] + g_ref[...]) * _dinv_of(degp_ref[...]) + b_ref[...]


def _final(sp, g, degp, b2):
    return pl.pallas_call(
        _final_body,
        grid=(NBLK,),
        in_specs=[
            pl.BlockSpec((NC, R, D), lambda i: (0, i, 0)),
            pl.BlockSpec((R, D), lambda i: (i, 0)),
            pl.BlockSpec((NW, R), lambda i: (0, i)),
            pl.BlockSpec((1, D), lambda i: (0, 0)),
        ],
        out_specs=pl.BlockSpec((R, D), lambda i: (i, 0)),
        out_shape=jax.ShapeDtypeStruct((N, D), jnp.float32),
    )(sp, g, degp, b2)


def kernel(x, edge_index, W1, b1, gamma, beta, W2, b2):
    src = edge_index[0].astype(jnp.int32)
    dst = edge_index[1].astype(jnp.int32)

    degp = _deg_kernel(dst)
    g1 = _mm_scale(x, W1, degp)
    sp1 = _agg_kernel(g1, src, dst)
    g2 = _bn_mm(sp1, g1, degp, b1.reshape(1, D), gamma.reshape(1, D),
                beta.reshape(1, D), W2)
    sp2 = _agg_kernel(g2, src, dst)
    return _final(sp2, g2, degp, b2.reshape(1, D))


# speedup vs baseline: 1.0371x; 1.0065x over previous
import jax
import jax.numpy as jnp
from jax import lax
from jax.experimental import pallas as pl
from jax.experimental.pallas import tpu as pltpu
from jax.experimental.pallas import tpu_sc as plsc

N = 10000
D = 128
E = 320000
EPS = 1e-5

NC = 2
NS = 16
NW = NC * NS
EPT = E // NW
CHUNK = 80
NCH = EPT // CHUNK
NPAD = 10240
RPT = NPAD // NS

_MESH = plsc.VectorSubcoreMesh(core_axis_name="c", subcore_axis_name="s")
_SC_PARAMS = pltpu.CompilerParams(needs_layout_passes=False)


def _deg_body(dst_hbm, degp_hbm, idx_v, hist_v):
    c = lax.axis_index("c")
    s = lax.axis_index("s")
    wid = c * NS + s
    zero16 = jnp.zeros((16,), jnp.float32)

    @pl.loop(0, NPAD // 16, unroll=8)
    def _(i):
        hist_v[pl.ds(i * 16, 16)] = zero16

    pltpu.sync_copy(dst_hbm.at[pl.ds(wid * EPT, EPT)], idx_v)
    ones16 = jnp.ones((16,), jnp.float32)

    @pl.loop(0, EPT // 16, unroll=16)
    def _(i):
        idx = idx_v[pl.ds(i * 16, 16)]
        plsc.addupdate_scatter(hist_v, [idx], ones16)

    pltpu.sync_copy(hist_v, degp_hbm.at[wid])


_deg_kernel = pl.kernel(
    _deg_body,
    out_type=jax.ShapeDtypeStruct((NW, NPAD), jnp.float32),
    mesh=_MESH,
    compiler_params=_SC_PARAMS,
    scratch_types=[
        pltpu.VMEM((EPT,), jnp.int32),
        pltpu.VMEM((NPAD,), jnp.float32),
    ],
)


NBUF = 3
NFULL = NCH // NBUF


def _agg_body(g_hbm, src_hbm, dst_hbm, out_hbm, sidx_v,
              didx_bufs, rows_bufs, acc_sh, sems):
    c = lax.axis_index("c")
    s = lax.axis_index("s")
    wid = c * NS + s
    zero16 = jnp.zeros((16,), jnp.float32)
    rows_a, sem_a = rows_bufs[0], sems[0]

    sidx_copy = pltpu.async_copy(src_hbm.at[pl.ds(wid * EPT, EPT)], sidx_v, sem_a)

    @pl.loop(0, CHUNK, unroll=4)
    def _(r):
        @pl.loop(0, D // 16)
        def _(j):
            rows_a[r, pl.ds(j * 16, 16)] = zero16

    @pl.loop(0, RPT // CHUNK)
    def _(j):
        pltpu.sync_copy(rows_a, acc_sh.at[pl.ds(s * RPT + j * CHUNK, CHUNK)])

    sidx_copy.wait()

    def start_chunk(k, j):
        pltpu.async_copy(g_hbm.at[sidx_v.at[pl.ds(k * CHUNK, CHUNK)]],
                         rows_bufs[j], sems[j])
        pltpu.async_copy(dst_hbm.at[pl.ds(wid * EPT + k * CHUNK, CHUNK)],
                         didx_bufs[j], sems[j])

    def wait_chunk(j):
        pltpu.make_async_copy(g_hbm.at[pl.ds(0, CHUNK)], rows_bufs[j], sems[j]).wait()
        pltpu.make_async_copy(dst_hbm.at[pl.ds(0, CHUNK)], didx_bufs[j], sems[j]).wait()

    def scatter(j):
        pltpu.sync_copy(rows_bufs[j], acc_sh.at[didx_bufs[j]], add=True)

    for j in range(NBUF):
        start_chunk(j, j)

    plsc.subcore_barrier()

    @pl.loop(0, NFULL)
    def _(i):
        k = i * NBUF
        for j in range(NBUF):
            cidx = k + j
            wait_chunk(j)
            scatter(j)

            @pl.when(cidx + NBUF < NCH)
            def _():
                start_chunk(cidx + NBUF, j)

    for j in range(NCH % NBUF):
        wait_chunk(j)
        scatter(j)

    plsc.subcore_barrier()

    pltpu.sync_copy(acc_sh.at[pl.ds(s * RPT, RPT)], out_hbm.at[c, pl.ds(s * RPT, RPT)])


_agg_kernel = pl.kernel(
    _agg_body,
    out_type=jax.ShapeDtypeStruct((NC, NPAD, D), jnp.float32),
    mesh=_MESH,
    compiler_params=_SC_PARAMS,
    scratch_types=[
        pltpu.VMEM((EPT,), jnp.int32),
        tuple(pltpu.VMEM((CHUNK,), jnp.int32) for _ in range(NBUF)),
        tuple(pltpu.VMEM((CHUNK, D), jnp.float32) for _ in range(NBUF)),
        pltpu.VMEM_SHARED((NPAD, D), jnp.float32),
        tuple(pltpu.SemaphoreType.DMA for _ in range(NBUF)),
    ],
)

R = 5120
NBLK = NPAD // R


def _dinv_of(degp_blk):
    deg = jnp.sum(degp_blk, axis=0) + 1.0
    return lax.rsqrt(deg)[:, None]


def _mm_scale_body(x_ref, w_ref, degp_ref, g_ref):
    h = jnp.dot(x_ref[...], w_ref[...], preferred_element_type=jnp.float32)
    g_ref[...] = h * _dinv_of(degp_ref[...])


def _mm_scale(x, w, degp):
    return pl.pallas_call(
        _mm_scale_body,
        grid=(NBLK,),
        in_specs=[
            pl.BlockSpec((R, D), lambda i: (i, 0)),
            pl.BlockSpec((D, D), lambda i: (0, 0)),
            pl.BlockSpec((NW, R), lambda i: (0, i)),
        ],
        out_specs=pl.BlockSpec((R, D), lambda i: (i, 0)),
        out_shape=jax.ShapeDtypeStruct((N, D), jnp.float32),
    )(x, w, degp)


def _bn_mm_body(sp_ref, g_ref, degp_ref, b_ref, gamma_ref, beta_ref, w_ref,
                g2_ref, t_v, st_v):
    p = pl.program_id(0)
    i = pl.program_id(1)
    dinv = _dinv_of(degp_ref[...])

    @pl.when(p == 0)
    def _():
        t = (sp_ref[0] + sp_ref[1] + g_ref[...]) * dinv + b_ref[...]
        t_v[pl.ds(i * R, R), :] = t
        rows = i * R + lax.broadcasted_iota(jnp.int32, (R, 1), 0)
        tm = jnp.where(rows < N, t, 0.0)
        st = jnp.concatenate(
            [jnp.sum(tm, axis=0, keepdims=True),
             jnp.sum(tm * tm, axis=0, keepdims=True)],
            axis=0,
        )

        @pl.when(i == 0)
        def _():
            st_v[...] = st

        @pl.when(i > 0)
        def _():
            st_v[...] = st_v[...] + st

    @pl.when(p == 1)
    def _():
        mean = st_v[0:1, :] * (1.0 / N)
        var = st_v[1:2, :] * (1.0 / N) - mean * mean
        y = gamma_ref[...] * (t_v[pl.ds(i * R, R), :] - mean) * lax.rsqrt(var + EPS) + beta_ref[...]
        y = jnp.maximum(y, 0.0)
        h2 = jnp.dot(y, w_ref[...], preferred_element_type=jnp.float32)
        g2_ref[...] = h2 * dinv


def _bn_mm(sp, g, degp, b1, gamma, beta, w2):
    return pl.pallas_call(
        _bn_mm_body,
        grid=(2, NBLK),
        in_specs=[
            pl.BlockSpec((NC, R, D), lambda p, i: (0, i * (1 - p), 0)),
            pl.BlockSpec((R, D), lambda p, i: (i * (1 - p), 0)),
            pl.BlockSpec((NW, R), lambda p, i: (0, i)),
            pl.BlockSpec((1, D), lambda p, i: (0, 0)),
            pl.BlockSpec((1, D), lambda p, i: (0, 0)),
            pl.BlockSpec((1, D), lambda p, i: (0, 0)),
            pl.BlockSpec((D, D), lambda p, i: (0, 0)),
        ],
        out_specs=pl.BlockSpec((R, D), lambda p, i: (i, 0)),
        out_shape=jax.ShapeDtypeStruct((N, D), jnp.float32),
        scratch_shapes=[
            pltpu.VMEM((NPAD, D), jnp.float32),
            pltpu.VMEM((2, D), jnp.float32),
        ],
    )(sp, g, degp, b1, gamma, beta, w2)


def _final_body(sp_ref, g_ref, degp_ref, b_ref, out_ref):
    out_ref[...] = (sp_ref[0] + sp_ref[1] + g_ref[...]) * _dinv_of(degp_ref[...]) + b_ref[...]


def _final(sp, g, degp, b2):
    return pl.pallas_call(
        _final_body,
        grid=(NBLK,),
        in_specs=[
            pl.BlockSpec((NC, R, D), lambda i: (0, i, 0)),
            pl.BlockSpec((R, D), lambda i: (i, 0)),
            pl.BlockSpec((NW, R), lambda i: (0, i)),
            pl.BlockSpec((1, D), lambda i: (0, 0)),
        ],
        out_specs=pl.BlockSpec((R, D), lambda i: (i, 0)),
        out_shape=jax.ShapeDtypeStruct((N, D), jnp.float32),
    )(sp, g, degp, b2)


def kernel(x, edge_index, W1, b1, gamma, beta, W2, b2):
    src = edge_index[0].astype(jnp.int32)
    dst = edge_index[1].astype(jnp.int32)

    degp = _deg_kernel(dst)
    g1 = _mm_scale(x, W1, degp)
    sp1 = _agg_kernel(g1, src, dst)
    g2 = _bn_mm(sp1, g1, degp, b1.reshape(1, D), gamma.reshape(1, D),
                beta.reshape(1, D), W2)
    sp2 = _agg_kernel(g2, src, dst)
    return _final(sp2, g2, degp, b2.reshape(1, D))
